# baseline (speedup 1.0000x reference)
import functools

import jax
import jax.numpy as jnp
from jax import lax
from jax.experimental import pallas as pl
from jax.experimental.pallas import tpu as pltpu
from jax.experimental.pallas import tpu_sc as plsc

_N = 10000
_E = 320000
_NW = 32
_NP = 10240
_R = _NP // _NW
_CH = 3200
_FLUSH = 1024
_STG = 2176
_CAP = 320 * 1024

_SC_PARAMS = pltpu.CompilerParams(needs_layout_passes=False)


def _mesh():
    return plsc.VectorSubcoreMesh(core_axis_name="c", subcore_axis_name="s")


def _wid():
    return lax.axis_index("s") * 2 + lax.axis_index("c")


def _bucket_body(dst_hbm, src_hbm, ew_hbm,
                 srcb_hbm, dstb_hbm, ewb_hbm, tot_hbm,
                 dstv, srcv, ewv, stg_s, stg_d, stg_w, totv, sm0, sm1):
    w = _wid()
    lo = w * _R
    dstb2 = (dstv.at[pl.ds(0, _CH)], dstv.at[pl.ds(_CH, _CH)])
    srcb2 = (srcv.at[pl.ds(0, _CH)], srcv.at[pl.ds(_CH, _CH)])
    ewb2 = (ewv.at[pl.ds(0, _CH)], ewv.at[pl.ds(_CH, _CH)])
    sems = (sm0, sm1)
    zi = jnp.zeros((16,), jnp.int32)
    zf = jnp.zeros((16,), jnp.float32)

    def fill(i, c):
        stg_s[pl.ds(i * 16, 16)] = zi
        stg_d[pl.ds(i * 16, 16)] = zi
        stg_w[pl.ds(i * 16, 16)] = zf
        return c

    lax.fori_loop(0, _STG // 16, fill, 0)

    def load_start(ch, b):
        base = ch * _CH
        pltpu.make_async_copy(dst_hbm.at[pl.ds(base, _CH)], dstb2[b],
                              sems[b]).start()
        pltpu.make_async_copy(src_hbm.at[pl.ds(base, _CH)], srcb2[b],
                              sems[b]).start()
        pltpu.make_async_copy(ew_hbm.at[pl.ds(base, _CH)], ewb2[b],
                              sems[b]).start()

    def load_wait(b):
        pltpu.make_async_copy(dst_hbm.at[pl.ds(0, _CH)], dstb2[b],
                              sems[b]).wait()
        pltpu.make_async_copy(src_hbm.at[pl.ds(0, _CH)], srcb2[b],
                              sems[b]).wait()
        pltpu.make_async_copy(ew_hbm.at[pl.ds(0, _CH)], ewb2[b],
                              sems[b]).wait()

    def flush(args):
        ptr, fl = args
        base = w * _CAP + fl * _FLUSH
        pltpu.sync_copy(stg_s.at[pl.ds(0, _FLUSH)],
                        srcb_hbm.at[pl.ds(base, _FLUSH)])
        pltpu.sync_copy(stg_d.at[pl.ds(0, _FLUSH)],
                        dstb_hbm.at[pl.ds(base, _FLUSH)])
        pltpu.sync_copy(stg_w.at[pl.ds(0, _FLUSH)],
                        ewb_hbm.at[pl.ds(base, _FLUSH)])
        for k in range(9):
            stg_s[pl.ds(k * 16, 16)] = stg_s[pl.ds(_FLUSH + k * 16, 16)]
            stg_d[pl.ds(k * 16, 16)] = stg_d[pl.ds(_FLUSH + k * 16, 16)]
            stg_w[pl.ds(k * 16, 16)] = stg_w[pl.ds(_FLUSH + k * 16, 16)]
        return ptr - _FLUSH, fl + 1

    def process(b, carry):
        def batch(bt, c2):
            ptr, fl = c2
            for j in range(8):
                off = (bt * 8 + j) * 16
                vd = dstb2[b][pl.ds(off, 16)]
                vs = srcb2[b][pl.ds(off, 16)]
                vw = ewb2[b][pl.ds(off, 16)]
                m = (vd >= lo) & (vd < lo + _R)
                plsc.store_compressed(stg_s.at[pl.ds(ptr, 16)], vs, mask=m)
                plsc.store_compressed(stg_d.at[pl.ds(ptr, 16)], vd - lo,
                                      mask=m)
                plsc.store_compressed(stg_w.at[pl.ds(ptr, 16)], vw, mask=m)
                ptr = ptr + plsc.all_reduce_population_count(m)[0]
            return lax.cond(ptr >= _FLUSH, flush, lambda a: a, (ptr, fl))

        return lax.fori_loop(0, _CH // 16 // 8, batch, carry)

    load_start(0, 0)

    def pair(i, carry):
        for b in range(2):
            ch = 2 * i + b
            load_wait(b)

            @pl.when(ch + 1 < _E // _CH)
            def _():
                load_start(ch + 1, 1 - b)

            carry = process(b, carry)
        return carry

    ptr, fl = lax.fori_loop(0, _E // _CH // 2, pair, (0, 0))
    base = w * _CAP + fl * _FLUSH
    pltpu.sync_copy(stg_s.at[pl.ds(0, _FLUSH)], srcb_hbm.at[pl.ds(base, _FLUSH)])
    pltpu.sync_copy(stg_d.at[pl.ds(0, _FLUSH)], dstb_hbm.at[pl.ds(base, _FLUSH)])
    pltpu.sync_copy(stg_w.at[pl.ds(0, _FLUSH)], ewb_hbm.at[pl.ds(base, _FLUSH)])
    totv[pl.ds(0, 16)] = jnp.zeros((16,), jnp.int32) + (fl + 1) * _FLUSH
    pltpu.sync_copy(totv, tot_hbm.at[w])


def _bucket(dst, src, ew):
    return pl.kernel(
        _bucket_body,
        out_type=[
            jax.ShapeDtypeStruct((_NW * _CAP,), jnp.int32),
            jax.ShapeDtypeStruct((_NW * _CAP,), jnp.int32),
            jax.ShapeDtypeStruct((_NW * _CAP,), jnp.float32),
            jax.ShapeDtypeStruct((_NW, 16), jnp.int32),
        ],
        mesh=_mesh(),
        scratch_types=[
            pltpu.VMEM((2 * _CH,), jnp.int32),
            pltpu.VMEM((2 * _CH,), jnp.int32),
            pltpu.VMEM((2 * _CH,), jnp.float32),
            pltpu.VMEM((_STG,), jnp.int32),
            pltpu.VMEM((_STG,), jnp.int32),
            pltpu.VMEM((_STG,), jnp.float32),
            pltpu.VMEM((16,), jnp.int32),
            pltpu.SemaphoreType.DMA,
            pltpu.SemaphoreType.DMA,
        ],
        compiler_params=_SC_PARAMS,
    )(dst, src, ew)


def _agg_body(D, G, hp_hbm, srcb_hbm, dstb_hbm, ewb_hbm, tot_hbm,
              neigh_hbm, idxv, dlv, ewv, rows0, rows1, acc, totv,
              sm0, sm1, sg0, sg1):
    w = _wid()
    shift = {64: 6, 128: 7}[G]
    pltpu.sync_copy(tot_hbm.at[w], totv)
    total = totv[pl.ds(0, 16)][0]
    nch = lax.shift_right_logical(total, shift)
    zf = jnp.zeros((16,), jnp.float32)

    def z(i, c):
        acc[pl.ds(i * 16, 16)] = zf
        return c

    lax.fori_loop(0, _R * D // 16, z, 0)

    idxb = (idxv.at[pl.ds(0, G)], idxv.at[pl.ds(G, G)])
    dlb = (dlv.at[pl.ds(0, G)], dlv.at[pl.ds(G, G)])
    ewb = (ewv.at[pl.ds(0, G)], ewv.at[pl.ds(G, G)])
    rowsb = (rows0, rows1)
    smb = (sm0, sm1)
    sgb = (sg0, sg1)

    def idx_start(ch, b):
        base = w * _CAP + ch * G
        pltpu.make_async_copy(srcb_hbm.at[pl.ds(base, G)], idxb[b],
                              smb[b]).start()

    def idx_wait(b):
        pltpu.make_async_copy(srcb_hbm.at[pl.ds(0, G)], idxb[b],
                              smb[b]).wait()

    def dlew_start(ch, b):
        base = w * _CAP + ch * G
        pltpu.make_async_copy(dstb_hbm.at[pl.ds(base, G)], dlb[b],
                              smb[b]).start()
        pltpu.make_async_copy(ewb_hbm.at[pl.ds(base, G)], ewb[b],
                              smb[b]).start()

    def dlew_wait(b):
        pltpu.make_async_copy(dstb_hbm.at[pl.ds(0, G)], dlb[b],
                              smb[b]).wait()
        pltpu.make_async_copy(ewb_hbm.at[pl.ds(0, G)], ewb[b],
                              smb[b]).wait()

    def gather_start(b):
        pltpu.make_async_copy(hp_hbm.at[idxb[b]], rowsb[b], sgb[b]).start()

    def gather_wait(b):
        pltpu.make_async_copy(hp_hbm.at[idxb[b]], rowsb[b], sgb[b]).wait()

    idx_start(0, 0)
    idx_wait(0)
    gather_start(0)
    idx_start(1, 1)
    dlew_start(0, 0)
    dlew_start(1, 1)

    def process(b):
        nc = D // 16
        ncb = min(nc, 8)

        def grp(g, c2):
            voff = dlb[b][pl.ds(g * 16, 16)] * D
            vew = ewb[b][pl.ds(g * 16, 16)]
            for j0 in range(0, 16, 4):
                ejs = [vew[j0 + j] for j in range(4)]
                abs_ = [voff[j0 + j] for j in range(4)]
                for c0 in range(0, nc, ncb):
                    mv = [[rowsb[b][g * 16 + j0 + j, pl.ds(c * 16, 16)]
                           * ejs[j] for c in range(c0, c0 + ncb)]
                          for j in range(4)]
                    for j in range(4):
                        ab = abs_[j]
                        cur = [acc[pl.ds(ab + c * 16, 16)]
                               for c in range(c0, c0 + ncb)]
                        for i, c in enumerate(range(c0, c0 + ncb)):
                            acc[pl.ds(ab + c * 16, 16)] = jnp.maximum(
                                cur[i], mv[j][i])
            return c2

        lax.fori_loop(0, G // 16, grp, 0)

    def pair(i, carry):
        for b in range(2):
            ch = 2 * i + b
            gather_wait(b)

            @pl.when(ch + 1 < nch)
            def _():
                idx_wait(1 - b)
                gather_start(1 - b)

            @pl.when(ch + 2 < nch)
            def _():
                idx_start(ch + 2, b)

            dlew_wait(b)
            process(b)

            @pl.when(ch + 2 < nch)
            def _():
                dlew_start(ch + 2, b)

        return carry

    lax.fori_loop(0, lax.shift_right_logical(nch, 1), pair, 0)
    pltpu.sync_copy(acc, neigh_hbm.at[pl.ds(w * _R * D, _R * D)])


@functools.lru_cache(maxsize=None)
def _make_agg(D):
    G = 128 if D <= 128 else 64
    return pl.kernel(
        functools.partial(_agg_body, D, G),
        out_type=jax.ShapeDtypeStruct((_NP * D,), jnp.float32),
        mesh=_mesh(),
        scratch_types=[
            pltpu.VMEM((2 * G,), jnp.int32),
            pltpu.VMEM((2 * G,), jnp.int32),
            pltpu.VMEM((2 * G,), jnp.float32),
            pltpu.VMEM((G, D), jnp.float32),
            pltpu.VMEM((G, D), jnp.float32),
            pltpu.VMEM((_R * D,), jnp.float32),
            pltpu.VMEM((16,), jnp.int32),
            pltpu.SemaphoreType.DMA,
            pltpu.SemaphoreType.DMA,
            pltpu.SemaphoreType.DMA,
            pltpu.SemaphoreType.DMA,
        ],
        compiler_params=_SC_PARAMS,
    )


def _mm1_body(x_ref, w_ref, b_ref, o_ref):
    y = lax.dot_general(x_ref[...], w_ref[...], (((1,), (1,)), ((), ())),
                        preferred_element_type=jnp.float32)
    o_ref[...] = jnp.maximum(y + b_ref[...], 0.0)


def _mm1(x, W, b):
    n, din = x.shape
    dout = W.shape[0]
    bm = 2048
    return pl.pallas_call(
        _mm1_body,
        grid=(n // bm,),
        in_specs=[pl.BlockSpec((bm, din), lambda i: (i, 0)),
                  pl.BlockSpec((dout, din), lambda i: (0, 0)),
                  pl.BlockSpec((1, dout), lambda i: (0, 0))],
        out_specs=pl.BlockSpec((bm, dout), lambda i: (i, 0)),
        out_shape=jax.ShapeDtypeStruct((n, dout), jnp.float32),
    )(x, W, b.reshape(1, -1))


def _mm2_body(x_ref, n_ref, ws_ref, wn_ref, b_ref, o_ref):
    y = lax.dot_general(x_ref[...], ws_ref[...], (((1,), (1,)), ((), ())),
                        preferred_element_type=jnp.float32)
    y = y + lax.dot_general(n_ref[...], wn_ref[...], (((1,), (1,)), ((), ())),
                            preferred_element_type=jnp.float32)
    o_ref[...] = jnp.maximum(y + b_ref[...], 0.0)


def _mm2(x, neigh, Ws, Wn, b):
    n, din = x.shape
    dout = Ws.shape[0]
    bm = 2048
    return pl.pallas_call(
        _mm2_body,
        grid=(n // bm,),
        in_specs=[pl.BlockSpec((bm, din), lambda i: (i, 0)),
                  pl.BlockSpec((bm, din), lambda i: (i, 0)),
                  pl.BlockSpec((dout, din), lambda i: (0, 0)),
                  pl.BlockSpec((dout, din), lambda i: (0, 0)),
                  pl.BlockSpec((1, dout), lambda i: (0, 0))],
        out_specs=pl.BlockSpec((bm, dout), lambda i: (i, 0)),
        out_shape=jax.ShapeDtypeStruct((n, dout), jnp.float32),
    )(x, neigh, Ws, Wn, b.reshape(1, -1))


def kernel(feat, edge_index, edge_weight,
           Wp1, bp1, Ws1, Wn1, b1,
           Wp2, bp2, Ws2, Wn2, b2,
           Wp3, bp3, Ws3, Wn3, b3):
    src = edge_index[0]
    dst = edge_index[1]
    srcb, dstb, ewb, tot = _bucket(dst, src, edge_weight)
    h = jnp.pad(feat, ((0, _NP - _N), (0, 0)))
    for Wp, bp, Ws, Wn, b in ((Wp1, bp1, Ws1, Wn1, b1),
                              (Wp2, bp2, Ws2, Wn2, b2),
                              (Wp3, bp3, Ws3, Wn3, b3)):
        d = Wp.shape[0]
        hp = _mm1(h, Wp, bp)
        neigh = _make_agg(d)(hp, srcb, dstb, ewb, tot).reshape(_NP, d)
        h = _mm2(h, neigh, Ws, Wn, b)
    return h[:_N]


# speedup vs baseline: 1.2072x; 1.2072x over previous
import functools

import jax
import jax.numpy as jnp
from jax import lax
from jax.experimental import pallas as pl
from jax.experimental.pallas import tpu as pltpu
from jax.experimental.pallas import tpu_sc as plsc

_N = 10000
_E = 320000
_NW = 32
_NP = 10240
_R = _NP // _NW
_CH = 3200
_FLUSH = 1024
_STG = 2176
_CAP = 320 * 1024

_SC_PARAMS = pltpu.CompilerParams(needs_layout_passes=False)


def _mesh():
    return plsc.VectorSubcoreMesh(core_axis_name="c", subcore_axis_name="s")


def _wid():
    return lax.axis_index("s") * 2 + lax.axis_index("c")


def _bucket_body(dst_hbm, src_hbm, ew_hbm,
                 srcb_hbm, dstb_hbm, ewb_hbm, tot_hbm,
                 dstv, srcv, ewv, stg_s, stg_d, stg_w, totv, sm0, sm1):
    w = _wid()
    lo = w * _R
    dstb2 = (dstv.at[pl.ds(0, _CH)], dstv.at[pl.ds(_CH, _CH)])
    srcb2 = (srcv.at[pl.ds(0, _CH)], srcv.at[pl.ds(_CH, _CH)])
    ewb2 = (ewv.at[pl.ds(0, _CH)], ewv.at[pl.ds(_CH, _CH)])
    sems = (sm0, sm1)
    zi = jnp.zeros((16,), jnp.int32)
    zf = jnp.zeros((16,), jnp.float32)

    def fill(i, c):
        stg_s[pl.ds(i * 16, 16)] = zi
        stg_d[pl.ds(i * 16, 16)] = zi
        stg_w[pl.ds(i * 16, 16)] = zf
        return c

    lax.fori_loop(0, _STG // 16, fill, 0)

    def load_start(ch, b):
        base = ch * _CH
        pltpu.make_async_copy(dst_hbm.at[pl.ds(base, _CH)], dstb2[b],
                              sems[b]).start()
        pltpu.make_async_copy(src_hbm.at[pl.ds(base, _CH)], srcb2[b],
                              sems[b]).start()
        pltpu.make_async_copy(ew_hbm.at[pl.ds(base, _CH)], ewb2[b],
                              sems[b]).start()

    def load_wait(b):
        pltpu.make_async_copy(dst_hbm.at[pl.ds(0, _CH)], dstb2[b],
                              sems[b]).wait()
        pltpu.make_async_copy(src_hbm.at[pl.ds(0, _CH)], srcb2[b],
                              sems[b]).wait()
        pltpu.make_async_copy(ew_hbm.at[pl.ds(0, _CH)], ewb2[b],
                              sems[b]).wait()

    def flush(args):
        ptr, fl = args
        base = w * _CAP + fl * _FLUSH
        pltpu.sync_copy(stg_s.at[pl.ds(0, _FLUSH)],
                        srcb_hbm.at[pl.ds(base, _FLUSH)])
        pltpu.sync_copy(stg_d.at[pl.ds(0, _FLUSH)],
                        dstb_hbm.at[pl.ds(base, _FLUSH)])
        pltpu.sync_copy(stg_w.at[pl.ds(0, _FLUSH)],
                        ewb_hbm.at[pl.ds(base, _FLUSH)])
        for k in range(9):
            stg_s[pl.ds(k * 16, 16)] = stg_s[pl.ds(_FLUSH + k * 16, 16)]
            stg_d[pl.ds(k * 16, 16)] = stg_d[pl.ds(_FLUSH + k * 16, 16)]
            stg_w[pl.ds(k * 16, 16)] = stg_w[pl.ds(_FLUSH + k * 16, 16)]
        return ptr - _FLUSH, fl + 1

    def process(b, carry):
        def batch(bt, c2):
            ptr, fl = c2
            vals = []
            for j in range(8):
                off = (bt * 8 + j) * 16
                vd = dstb2[b][pl.ds(off, 16)]
                vs = srcb2[b][pl.ds(off, 16)]
                vw = ewb2[b][pl.ds(off, 16)]
                m = (vd >= lo) & (vd < lo + _R)
                cnt = plsc.all_reduce_population_count(m)[0]
                vals.append((vd - lo, vs, vw, m, cnt))
            for vdl, vs, vw, m, cnt in vals:
                plsc.store_compressed(stg_s.at[pl.ds(ptr, 16)], vs, mask=m)
                plsc.store_compressed(stg_d.at[pl.ds(ptr, 16)], vdl, mask=m)
                plsc.store_compressed(stg_w.at[pl.ds(ptr, 16)], vw, mask=m)
                ptr = ptr + cnt
            return lax.cond(ptr >= _FLUSH, flush, lambda a: a, (ptr, fl))

        return lax.fori_loop(0, _CH // 16 // 8, batch, carry)

    load_start(0, 0)

    def pair(i, carry):
        for b in range(2):
            ch = 2 * i + b
            load_wait(b)

            @pl.when(ch + 1 < _E // _CH)
            def _():
                load_start(ch + 1, 1 - b)

            carry = process(b, carry)
        return carry

    ptr, fl = lax.fori_loop(0, _E // _CH // 2, pair, (0, 0))
    base = w * _CAP + fl * _FLUSH
    pltpu.sync_copy(stg_s.at[pl.ds(0, _FLUSH)], srcb_hbm.at[pl.ds(base, _FLUSH)])
    pltpu.sync_copy(stg_d.at[pl.ds(0, _FLUSH)], dstb_hbm.at[pl.ds(base, _FLUSH)])
    pltpu.sync_copy(stg_w.at[pl.ds(0, _FLUSH)], ewb_hbm.at[pl.ds(base, _FLUSH)])
    totv[pl.ds(0, 16)] = jnp.zeros((16,), jnp.int32) + (fl + 1) * _FLUSH
    pltpu.sync_copy(totv, tot_hbm.at[w])


def _bucket(dst, src, ew):
    return pl.kernel(
        _bucket_body,
        out_type=[
            jax.ShapeDtypeStruct((_NW * _CAP,), jnp.int32),
            jax.ShapeDtypeStruct((_NW * _CAP,), jnp.int32),
            jax.ShapeDtypeStruct((_NW * _CAP,), jnp.float32),
            jax.ShapeDtypeStruct((_NW, 16), jnp.int32),
        ],
        mesh=_mesh(),
        scratch_types=[
            pltpu.VMEM((2 * _CH,), jnp.int32),
            pltpu.VMEM((2 * _CH,), jnp.int32),
            pltpu.VMEM((2 * _CH,), jnp.float32),
            pltpu.VMEM((_STG,), jnp.int32),
            pltpu.VMEM((_STG,), jnp.int32),
            pltpu.VMEM((_STG,), jnp.float32),
            pltpu.VMEM((16,), jnp.int32),
            pltpu.SemaphoreType.DMA,
            pltpu.SemaphoreType.DMA,
        ],
        compiler_params=_SC_PARAMS,
    )(dst, src, ew)


def _agg_body(D, G, hp_hbm, srcb_hbm, dstb_hbm, ewb_hbm, tot_hbm,
              neigh_hbm, idxv, dlv, ewv, rows0, rows1, acc, totv,
              sm0, sm1, sg0, sg1):
    w = _wid()
    shift = {64: 6, 128: 7}[G]
    pltpu.sync_copy(tot_hbm.at[w], totv)
    total = totv[pl.ds(0, 16)][0]
    nch = lax.shift_right_logical(total, shift)
    zf = jnp.zeros((16,), jnp.float32)

    def z(i, c):
        acc[pl.ds(i * 16, 16)] = zf
        return c

    lax.fori_loop(0, _R * D // 16, z, 0)

    idxb = (idxv.at[pl.ds(0, G)], idxv.at[pl.ds(G, G)])
    dlb = (dlv.at[pl.ds(0, G)], dlv.at[pl.ds(G, G)])
    ewb = (ewv.at[pl.ds(0, G)], ewv.at[pl.ds(G, G)])
    rowsb = (rows0, rows1)
    smb = (sm0, sm1)
    sgb = (sg0, sg1)

    def idx_start(ch, b):
        base = w * _CAP + ch * G
        pltpu.make_async_copy(srcb_hbm.at[pl.ds(base, G)], idxb[b],
                              smb[b]).start()

    def idx_wait(b):
        pltpu.make_async_copy(srcb_hbm.at[pl.ds(0, G)], idxb[b],
                              smb[b]).wait()

    def dlew_start(ch, b):
        base = w * _CAP + ch * G
        pltpu.make_async_copy(dstb_hbm.at[pl.ds(base, G)], dlb[b],
                              smb[b]).start()
        pltpu.make_async_copy(ewb_hbm.at[pl.ds(base, G)], ewb[b],
                              smb[b]).start()

    def dlew_wait(b):
        pltpu.make_async_copy(dstb_hbm.at[pl.ds(0, G)], dlb[b],
                              smb[b]).wait()
        pltpu.make_async_copy(ewb_hbm.at[pl.ds(0, G)], ewb[b],
                              smb[b]).wait()

    def gather_start(b):
        pltpu.make_async_copy(hp_hbm.at[idxb[b]], rowsb[b], sgb[b]).start()

    def gather_wait(b):
        pltpu.make_async_copy(hp_hbm.at[idxb[b]], rowsb[b], sgb[b]).wait()

    idx_start(0, 0)
    idx_wait(0)
    gather_start(0)
    idx_start(1, 1)
    dlew_start(0, 0)
    dlew_start(1, 1)

    def process(b):
        nc = D // 16

        def grp(g, c2):
            voff = dlb[b][pl.ds(g * 16, 16)] * D
            vew = ewb[b][pl.ds(g * 16, 16)]
            for j in range(16):
                ej = vew[j]
                r = g * 16 + j
                ab = voff[j]
                mv = [rowsb[b][r, pl.ds(c * 16, 16)] * ej for c in range(nc)]
                cur = [acc[pl.ds(ab + c * 16, 16)] for c in range(nc)]
                for c in range(nc):
                    acc[pl.ds(ab + c * 16, 16)] = jnp.maximum(cur[c], mv[c])
            return c2

        lax.fori_loop(0, G // 16, grp, 0)

    def pair(i, carry):
        for b in range(2):
            ch = 2 * i + b
            gather_wait(b)

            @pl.when(ch + 1 < nch)
            def _():
                idx_wait(1 - b)
                gather_start(1 - b)

            @pl.when(ch + 2 < nch)
            def _():
                idx_start(ch + 2, b)

            dlew_wait(b)
            process(b)

            @pl.when(ch + 2 < nch)
            def _():
                dlew_start(ch + 2, b)

        return carry

    lax.fori_loop(0, lax.shift_right_logical(nch, 1), pair, 0)
    pltpu.sync_copy(acc, neigh_hbm.at[pl.ds(w * _R * D, _R * D)])


@functools.lru_cache(maxsize=None)
def _make_agg(D):
    G = 128 if D <= 128 else 64
    return pl.kernel(
        functools.partial(_agg_body, D, G),
        out_type=jax.ShapeDtypeStruct((_NP * D,), jnp.float32),
        mesh=_mesh(),
        scratch_types=[
            pltpu.VMEM((2 * G,), jnp.int32),
            pltpu.VMEM((2 * G,), jnp.int32),
            pltpu.VMEM((2 * G,), jnp.float32),
            pltpu.VMEM((G, D), jnp.float32),
            pltpu.VMEM((G, D), jnp.float32),
            pltpu.VMEM((_R * D,), jnp.float32),
            pltpu.VMEM((16,), jnp.int32),
            pltpu.SemaphoreType.DMA,
            pltpu.SemaphoreType.DMA,
            pltpu.SemaphoreType.DMA,
            pltpu.SemaphoreType.DMA,
        ],
        compiler_params=_SC_PARAMS,
    )


def _mm1_body(x_ref, w_ref, b_ref, o_ref):
    y = lax.dot_general(x_ref[...], w_ref[...], (((1,), (1,)), ((), ())),
                        preferred_element_type=jnp.float32)
    o_ref[...] = jnp.maximum(y + b_ref[...], 0.0)


def _mm1(x, W, b):
    n, din = x.shape
    dout = W.shape[0]
    bm = 2048
    return pl.pallas_call(
        _mm1_body,
        grid=(n // bm,),
        in_specs=[pl.BlockSpec((bm, din), lambda i: (i, 0)),
                  pl.BlockSpec((dout, din), lambda i: (0, 0)),
                  pl.BlockSpec((1, dout), lambda i: (0, 0))],
        out_specs=pl.BlockSpec((bm, dout), lambda i: (i, 0)),
        out_shape=jax.ShapeDtypeStruct((n, dout), jnp.float32),
    )(x, W, b.reshape(1, -1))


def _mm2_body(x_ref, n_ref, ws_ref, wn_ref, b_ref, o_ref):
    y = lax.dot_general(x_ref[...], ws_ref[...], (((1,), (1,)), ((), ())),
                        preferred_element_type=jnp.float32)
    y = y + lax.dot_general(n_ref[...], wn_ref[...], (((1,), (1,)), ((), ())),
                            preferred_element_type=jnp.float32)
    o_ref[...] = jnp.maximum(y + b_ref[...], 0.0)


def _mm2(x, neigh, Ws, Wn, b):
    n, din = x.shape
    dout = Ws.shape[0]
    bm = 2048
    return pl.pallas_call(
        _mm2_body,
        grid=(n // bm,),
        in_specs=[pl.BlockSpec((bm, din), lambda i: (i, 0)),
                  pl.BlockSpec((bm, din), lambda i: (i, 0)),
                  pl.BlockSpec((dout, din), lambda i: (0, 0)),
                  pl.BlockSpec((dout, din), lambda i: (0, 0)),
                  pl.BlockSpec((1, dout), lambda i: (0, 0))],
        out_specs=pl.BlockSpec((bm, dout), lambda i: (i, 0)),
        out_shape=jax.ShapeDtypeStruct((n, dout), jnp.float32),
    )(x, neigh, Ws, Wn, b.reshape(1, -1))


def kernel(feat, edge_index, edge_weight,
           Wp1, bp1, Ws1, Wn1, b1,
           Wp2, bp2, Ws2, Wn2, b2,
           Wp3, bp3, Ws3, Wn3, b3):
    src = edge_index[0]
    dst = edge_index[1]
    srcb, dstb, ewb, tot = _bucket(dst, src, edge_weight)
    h = jnp.pad(feat, ((0, _NP - _N), (0, 0)))
    for Wp, bp, Ws, Wn, b in ((Wp1, bp1, Ws1, Wn1, b1),
                              (Wp2, bp2, Ws2, Wn2, b2),
                              (Wp3, bp3, Ws3, Wn3, b3)):
        d = Wp.shape[0]
        hp = _mm1(h, Wp, bp)
        neigh = _make_agg(d)(hp, srcb, dstb, ewb, tot).reshape(_NP, d)
        h = _mm2(h, neigh, Ws, Wn, b)
    return h[:_N]


# speedup vs baseline: 1.2609x; 1.0445x over previous
import functools

import jax
import jax.numpy as jnp
from jax import lax
from jax.experimental import pallas as pl
from jax.experimental.pallas import tpu as pltpu
from jax.experimental.pallas import tpu_sc as plsc

_N = 10000
_E = 320000
_NW = 32
_NP = 10240
_R = _NP // _NW
_CH = 3200
_FLUSH = 1024
_STG = 2176
_CAP = 320 * 1024

_SC_PARAMS = pltpu.CompilerParams(needs_layout_passes=False)


def _mesh():
    return plsc.VectorSubcoreMesh(core_axis_name="c", subcore_axis_name="s")


def _wid():
    return lax.axis_index("s") * 2 + lax.axis_index("c")


def _bucket_body(dst_hbm, src_hbm, ew_hbm,
                 srcb_hbm, dstb_hbm, ewb_hbm, tot_hbm,
                 dstv, srcv, ewv, stg_s, stg_d, stg_w, totv, sm0, sm1):
    w = _wid()
    lo = w * _R
    dstb2 = (dstv.at[pl.ds(0, _CH)], dstv.at[pl.ds(_CH, _CH)])
    srcb2 = (srcv.at[pl.ds(0, _CH)], srcv.at[pl.ds(_CH, _CH)])
    ewb2 = (ewv.at[pl.ds(0, _CH)], ewv.at[pl.ds(_CH, _CH)])
    sems = (sm0, sm1)
    zi = jnp.zeros((16,), jnp.int32)
    zf = jnp.zeros((16,), jnp.float32)

    def fill(i, c):
        stg_s[pl.ds(i * 16, 16)] = zi
        stg_d[pl.ds(i * 16, 16)] = zi
        stg_w[pl.ds(i * 16, 16)] = zf
        return c

    lax.fori_loop(0, _STG // 16, fill, 0)

    def load_start(ch, b):
        base = ch * _CH
        pltpu.make_async_copy(dst_hbm.at[pl.ds(base, _CH)], dstb2[b],
                              sems[b]).start()
        pltpu.make_async_copy(src_hbm.at[pl.ds(base, _CH)], srcb2[b],
                              sems[b]).start()
        pltpu.make_async_copy(ew_hbm.at[pl.ds(base, _CH)], ewb2[b],
                              sems[b]).start()

    def load_wait(b):
        pltpu.make_async_copy(dst_hbm.at[pl.ds(0, _CH)], dstb2[b],
                              sems[b]).wait()
        pltpu.make_async_copy(src_hbm.at[pl.ds(0, _CH)], srcb2[b],
                              sems[b]).wait()
        pltpu.make_async_copy(ew_hbm.at[pl.ds(0, _CH)], ewb2[b],
                              sems[b]).wait()

    def flush(args):
        ptr, fl = args
        base = w * _CAP + fl * _FLUSH
        pltpu.sync_copy(stg_s.at[pl.ds(0, _FLUSH)],
                        srcb_hbm.at[pl.ds(base, _FLUSH)])
        pltpu.sync_copy(stg_d.at[pl.ds(0, _FLUSH)],
                        dstb_hbm.at[pl.ds(base, _FLUSH)])
        pltpu.sync_copy(stg_w.at[pl.ds(0, _FLUSH)],
                        ewb_hbm.at[pl.ds(base, _FLUSH)])
        for k in range(9):
            stg_s[pl.ds(k * 16, 16)] = stg_s[pl.ds(_FLUSH + k * 16, 16)]
            stg_d[pl.ds(k * 16, 16)] = stg_d[pl.ds(_FLUSH + k * 16, 16)]
            stg_w[pl.ds(k * 16, 16)] = stg_w[pl.ds(_FLUSH + k * 16, 16)]
        return ptr - _FLUSH, fl + 1

    def process(b, carry):
        def batch(bt, c2):
            ptr, fl = c2
            vals = []
            for j in range(8):
                off = (bt * 8 + j) * 16
                vd = dstb2[b][pl.ds(off, 16)]
                vs = srcb2[b][pl.ds(off, 16)]
                vw = ewb2[b][pl.ds(off, 16)]
                m = (vd >= lo) & (vd < lo + _R)
                cnt = plsc.all_reduce_population_count(m)[0]
                vals.append((vd - lo, vs, vw, m, cnt))
            for vdl, vs, vw, m, cnt in vals:
                plsc.store_compressed(stg_s.at[pl.ds(ptr, 16)], vs, mask=m)
                plsc.store_compressed(stg_d.at[pl.ds(ptr, 16)], vdl, mask=m)
                plsc.store_compressed(stg_w.at[pl.ds(ptr, 16)], vw, mask=m)
                ptr = ptr + cnt
            return lax.cond(ptr >= _FLUSH, flush, lambda a: a, (ptr, fl))

        return lax.fori_loop(0, _CH // 16 // 8, batch, carry)

    load_start(0, 0)

    def pair(i, carry):
        for b in range(2):
            ch = 2 * i + b
            load_wait(b)

            @pl.when(ch + 1 < _E // _CH)
            def _():
                load_start(ch + 1, 1 - b)

            carry = process(b, carry)
        return carry

    ptr, fl = lax.fori_loop(0, _E // _CH // 2, pair, (0, 0))
    base = w * _CAP + fl * _FLUSH
    pltpu.sync_copy(stg_s.at[pl.ds(0, _FLUSH)], srcb_hbm.at[pl.ds(base, _FLUSH)])
    pltpu.sync_copy(stg_d.at[pl.ds(0, _FLUSH)], dstb_hbm.at[pl.ds(base, _FLUSH)])
    pltpu.sync_copy(stg_w.at[pl.ds(0, _FLUSH)], ewb_hbm.at[pl.ds(base, _FLUSH)])
    totv[pl.ds(0, 16)] = jnp.zeros((16,), jnp.int32) + (fl + 1) * _FLUSH
    pltpu.sync_copy(totv, tot_hbm.at[w])


def _bucket(dst, src, ew):
    return pl.kernel(
        _bucket_body,
        out_type=[
            jax.ShapeDtypeStruct((_NW * _CAP,), jnp.int32),
            jax.ShapeDtypeStruct((_NW * _CAP,), jnp.int32),
            jax.ShapeDtypeStruct((_NW * _CAP,), jnp.float32),
            jax.ShapeDtypeStruct((_NW, 16), jnp.int32),
        ],
        mesh=_mesh(),
        scratch_types=[
            pltpu.VMEM((2 * _CH,), jnp.int32),
            pltpu.VMEM((2 * _CH,), jnp.int32),
            pltpu.VMEM((2 * _CH,), jnp.float32),
            pltpu.VMEM((_STG,), jnp.int32),
            pltpu.VMEM((_STG,), jnp.int32),
            pltpu.VMEM((_STG,), jnp.float32),
            pltpu.VMEM((16,), jnp.int32),
            pltpu.SemaphoreType.DMA,
            pltpu.SemaphoreType.DMA,
        ],
        compiler_params=_SC_PARAMS,
    )(dst, src, ew)


def _agg_body(D, G, hp_hbm, srcb_hbm, dstb_hbm, ewb_hbm, tot_hbm,
              neigh_hbm, idxv, dlv, ewv, rows0, rows1, acc, totv,
              sm0, sm1, sg0, sg1):
    w = _wid()
    shift = {64: 6, 128: 7}[G]
    pltpu.sync_copy(tot_hbm.at[w], totv)
    total = totv[pl.ds(0, 16)][0]
    nch = lax.shift_right_logical(total, shift)
    zf = jnp.zeros((16,), jnp.float32)

    idxb = (idxv.at[pl.ds(0, G)], idxv.at[pl.ds(G, G)])
    dlb = (dlv.at[pl.ds(0, G)], dlv.at[pl.ds(G, G)])
    ewb = (ewv.at[pl.ds(0, G)], ewv.at[pl.ds(G, G)])
    rowsb = (rows0, rows1)
    smb = (sm0, sm1)
    sgb = (sg0, sg1)

    def idx_start(ch, b):
        base = w * _CAP + ch * G
        pltpu.make_async_copy(srcb_hbm.at[pl.ds(base, G)], idxb[b],
                              smb[b]).start()

    def idx_wait(b):
        pltpu.make_async_copy(srcb_hbm.at[pl.ds(0, G)], idxb[b],
                              smb[b]).wait()

    def dlew_start(ch, b):
        base = w * _CAP + ch * G
        pltpu.make_async_copy(dstb_hbm.at[pl.ds(base, G)], dlb[b],
                              smb[b]).start()
        pltpu.make_async_copy(ewb_hbm.at[pl.ds(base, G)], ewb[b],
                              smb[b]).start()

    def dlew_wait(b):
        pltpu.make_async_copy(dstb_hbm.at[pl.ds(0, G)], dlb[b],
                              smb[b]).wait()
        pltpu.make_async_copy(ewb_hbm.at[pl.ds(0, G)], ewb[b],
                              smb[b]).wait()

    def gather_start(b):
        pltpu.make_async_copy(hp_hbm.at[idxb[b]], rowsb[b], sgb[b]).start()

    def gather_wait(b):
        pltpu.make_async_copy(hp_hbm.at[idxb[b]], rowsb[b], sgb[b]).wait()

    idx_start(0, 0)
    dlew_start(0, 0)
    idx_start(1, 1)
    dlew_start(1, 1)

    def z(i, c):
        for k in range(8):
            acc[pl.ds((i * 8 + k) * 16, 16)] = zf
        return c

    lax.fori_loop(0, _R * D // 128, z, 0)
    idx_wait(0)
    gather_start(0)

    def process(b):
        nc = D // 16

        def grp(g, c2):
            voff = dlb[b][pl.ds(g * 16, 16)] * D
            vew = ewb[b][pl.ds(g * 16, 16)]
            for j in range(16):
                ej = vew[j]
                r = g * 16 + j
                ab = voff[j]
                mv = [rowsb[b][r, pl.ds(c * 16, 16)] * ej for c in range(nc)]
                cur = [acc[pl.ds(ab + c * 16, 16)] for c in range(nc)]
                for c in range(nc):
                    acc[pl.ds(ab + c * 16, 16)] = jnp.maximum(cur[c], mv[c])
            return c2

        lax.fori_loop(0, G // 16, grp, 0)

    def pair(i, carry):
        for b in range(2):
            ch = 2 * i + b
            gather_wait(b)

            @pl.when(ch + 1 < nch)
            def _():
                idx_wait(1 - b)
                gather_start(1 - b)

            @pl.when(ch + 2 < nch)
            def _():
                idx_start(ch + 2, b)

            dlew_wait(b)
            process(b)

            @pl.when(ch + 2 < nch)
            def _():
                dlew_start(ch + 2, b)

        return carry

    lax.fori_loop(0, lax.shift_right_logical(nch, 1), pair, 0)
    pltpu.sync_copy(acc, neigh_hbm.at[pl.ds(w * _R * D, _R * D)])


@functools.lru_cache(maxsize=None)
def _make_agg(D):
    G = 128 if D <= 128 else 64
    return pl.kernel(
        functools.partial(_agg_body, D, G),
        out_type=jax.ShapeDtypeStruct((_NP * D,), jnp.float32),
        mesh=_mesh(),
        scratch_types=[
            pltpu.VMEM((2 * G,), jnp.int32),
            pltpu.VMEM((2 * G,), jnp.int32),
            pltpu.VMEM((2 * G,), jnp.float32),
            pltpu.VMEM((G, D), jnp.float32),
            pltpu.VMEM((G, D), jnp.float32),
            pltpu.VMEM((_R * D,), jnp.float32),
            pltpu.VMEM((16,), jnp.int32),
            pltpu.SemaphoreType.DMA,
            pltpu.SemaphoreType.DMA,
            pltpu.SemaphoreType.DMA,
            pltpu.SemaphoreType.DMA,
        ],
        compiler_params=_SC_PARAMS,
    )


def _mm1_body(x_ref, w_ref, b_ref, o_ref):
    y = lax.dot_general(x_ref[...], w_ref[...], (((1,), (1,)), ((), ())),
                        preferred_element_type=jnp.float32)
    o_ref[...] = jnp.maximum(y + b_ref[...], 0.0)


def _mm1(x, W, b):
    n, din = x.shape
    dout = W.shape[0]
    bm = 2048
    return pl.pallas_call(
        _mm1_body,
        grid=(n // bm,),
        in_specs=[pl.BlockSpec((bm, din), lambda i: (i, 0)),
                  pl.BlockSpec((dout, din), lambda i: (0, 0)),
                  pl.BlockSpec((1, dout), lambda i: (0, 0))],
        out_specs=pl.BlockSpec((bm, dout), lambda i: (i, 0)),
        out_shape=jax.ShapeDtypeStruct((n, dout), jnp.float32),
    )(x, W, b.reshape(1, -1))


def _mm2_body(x_ref, n_ref, ws_ref, wn_ref, b_ref, o_ref):
    y = lax.dot_general(x_ref[...], ws_ref[...], (((1,), (1,)), ((), ())),
                        preferred_element_type=jnp.float32)
    y = y + lax.dot_general(n_ref[...], wn_ref[...], (((1,), (1,)), ((), ())),
                            preferred_element_type=jnp.float32)
    o_ref[...] = jnp.maximum(y + b_ref[...], 0.0)


def _mm2(x, neigh, Ws, Wn, b):
    n, din = x.shape
    dout = Ws.shape[0]
    bm = 2048
    return pl.pallas_call(
        _mm2_body,
        grid=(n // bm,),
        in_specs=[pl.BlockSpec((bm, din), lambda i: (i, 0)),
                  pl.BlockSpec((bm, din), lambda i: (i, 0)),
                  pl.BlockSpec((dout, din), lambda i: (0, 0)),
                  pl.BlockSpec((dout, din), lambda i: (0, 0)),
                  pl.BlockSpec((1, dout), lambda i: (0, 0))],
        out_specs=pl.BlockSpec((bm, dout), lambda i: (i, 0)),
        out_shape=jax.ShapeDtypeStruct((n, dout), jnp.float32),
    )(x, neigh, Ws, Wn, b.reshape(1, -1))


def kernel(feat, edge_index, edge_weight,
           Wp1, bp1, Ws1, Wn1, b1,
           Wp2, bp2, Ws2, Wn2, b2,
           Wp3, bp3, Ws3, Wn3, b3):
    src = edge_index[0]
    dst = edge_index[1]
    srcb, dstb, ewb, tot = _bucket(dst, src, edge_weight)
    h = jnp.pad(feat, ((0, _NP - _N), (0, 0)))
    for Wp, bp, Ws, Wn, b in ((Wp1, bp1, Ws1, Wn1, b1),
                              (Wp2, bp2, Ws2, Wn2, b2),
                              (Wp3, bp3, Ws3, Wn3, b3)):
        d = Wp.shape[0]
        hp = _mm1(h, Wp, bp)
        neigh = _make_agg(d)(hp, srcb, dstb, ewb, tot).reshape(_NP, d)
        h = _mm2(h, neigh, Ws, Wn, b)
    return h[:_N]


# speedup vs baseline: 1.2798x; 1.0150x over previous
import functools

import jax
import jax.numpy as jnp
from jax import lax
from jax.experimental import pallas as pl
from jax.experimental.pallas import tpu as pltpu
from jax.experimental.pallas import tpu_sc as plsc

_N = 10000
_E = 320000
_NW = 32
_NP = 10240
_R = _NP // _NW
_CH = 3200
_FLUSH = 1024
_STG = 2176
_CAP = 320 * 1024

_SC_PARAMS = pltpu.CompilerParams(needs_layout_passes=False)


def _mesh():
    return plsc.VectorSubcoreMesh(core_axis_name="c", subcore_axis_name="s")


def _wid():
    return lax.axis_index("s") * 2 + lax.axis_index("c")


def _bucket_body(dst_hbm, src_hbm, ew_hbm,
                 srcb_hbm, dstb_hbm, ewb_hbm, tot_hbm,
                 dstv, srcv, ewv, stg_s, stg_d, stg_w, totv, sm0, sm1):
    w = _wid()
    lo = w * _R
    dstb2 = (dstv.at[pl.ds(0, _CH)], dstv.at[pl.ds(_CH, _CH)])
    srcb2 = (srcv.at[pl.ds(0, _CH)], srcv.at[pl.ds(_CH, _CH)])
    ewb2 = (ewv.at[pl.ds(0, _CH)], ewv.at[pl.ds(_CH, _CH)])
    sems = (sm0, sm1)
    zi = jnp.zeros((16,), jnp.int32)
    zf = jnp.zeros((16,), jnp.float32)

    def fill(i, c):
        stg_s[pl.ds(i * 16, 16)] = zi
        stg_d[pl.ds(i * 16, 16)] = zi
        stg_w[pl.ds(i * 16, 16)] = zf
        return c

    lax.fori_loop(0, _STG // 16, fill, 0)

    def load_start(ch, b):
        base = ch * _CH
        pltpu.make_async_copy(dst_hbm.at[pl.ds(base, _CH)], dstb2[b],
                              sems[b]).start()
        pltpu.make_async_copy(src_hbm.at[pl.ds(base, _CH)], srcb2[b],
                              sems[b]).start()
        pltpu.make_async_copy(ew_hbm.at[pl.ds(base, _CH)], ewb2[b],
                              sems[b]).start()

    def load_wait(b):
        pltpu.make_async_copy(dst_hbm.at[pl.ds(0, _CH)], dstb2[b],
                              sems[b]).wait()
        pltpu.make_async_copy(src_hbm.at[pl.ds(0, _CH)], srcb2[b],
                              sems[b]).wait()
        pltpu.make_async_copy(ew_hbm.at[pl.ds(0, _CH)], ewb2[b],
                              sems[b]).wait()

    def flush(args):
        ptr, fl = args
        base = w * _CAP + fl * _FLUSH
        pltpu.sync_copy(stg_s.at[pl.ds(0, _FLUSH)],
                        srcb_hbm.at[pl.ds(base, _FLUSH)])
        pltpu.sync_copy(stg_d.at[pl.ds(0, _FLUSH)],
                        dstb_hbm.at[pl.ds(base, _FLUSH)])
        pltpu.sync_copy(stg_w.at[pl.ds(0, _FLUSH)],
                        ewb_hbm.at[pl.ds(base, _FLUSH)])
        for k in range(9):
            stg_s[pl.ds(k * 16, 16)] = stg_s[pl.ds(_FLUSH + k * 16, 16)]
            stg_d[pl.ds(k * 16, 16)] = stg_d[pl.ds(_FLUSH + k * 16, 16)]
            stg_w[pl.ds(k * 16, 16)] = stg_w[pl.ds(_FLUSH + k * 16, 16)]
        return ptr - _FLUSH, fl + 1

    def process(b, carry):
        def batch(bt, c2):
            ptr, fl = c2
            vals = []
            for j in range(8):
                off = (bt * 8 + j) * 16
                vd = dstb2[b][pl.ds(off, 16)]
                vs = srcb2[b][pl.ds(off, 16)]
                vw = ewb2[b][pl.ds(off, 16)]
                m = (vd >= lo) & (vd < lo + _R)
                cnt = plsc.all_reduce_population_count(m)[0]
                vals.append((vd - lo, vs, vw, m, cnt))
            for vdl, vs, vw, m, cnt in vals:
                plsc.store_compressed(stg_s.at[pl.ds(ptr, 16)], vs, mask=m)
                plsc.store_compressed(stg_d.at[pl.ds(ptr, 16)], vdl, mask=m)
                plsc.store_compressed(stg_w.at[pl.ds(ptr, 16)], vw, mask=m)
                ptr = ptr + cnt
            return lax.cond(ptr >= _FLUSH, flush, lambda a: a, (ptr, fl))

        return lax.fori_loop(0, _CH // 16 // 8, batch, carry)

    load_start(0, 0)

    def pair(i, carry):
        for b in range(2):
            ch = 2 * i + b
            load_wait(b)

            @pl.when(ch + 1 < _E // _CH)
            def _():
                load_start(ch + 1, 1 - b)

            carry = process(b, carry)
        return carry

    ptr, fl = lax.fori_loop(0, _E // _CH // 2, pair, (0, 0))
    base = w * _CAP + fl * _FLUSH
    pltpu.sync_copy(stg_s.at[pl.ds(0, _FLUSH)], srcb_hbm.at[pl.ds(base, _FLUSH)])
    pltpu.sync_copy(stg_d.at[pl.ds(0, _FLUSH)], dstb_hbm.at[pl.ds(base, _FLUSH)])
    pltpu.sync_copy(stg_w.at[pl.ds(0, _FLUSH)], ewb_hbm.at[pl.ds(base, _FLUSH)])
    totv[pl.ds(0, 16)] = jnp.zeros((16,), jnp.int32) + (fl + 1) * _FLUSH
    pltpu.sync_copy(totv, tot_hbm.at[w])


def _bucket(dst, src, ew):
    return pl.kernel(
        _bucket_body,
        out_type=[
            jax.ShapeDtypeStruct((_NW * _CAP,), jnp.int32),
            jax.ShapeDtypeStruct((_NW * _CAP,), jnp.int32),
            jax.ShapeDtypeStruct((_NW * _CAP,), jnp.float32),
            jax.ShapeDtypeStruct((_NW, 16), jnp.int32),
        ],
        mesh=_mesh(),
        scratch_types=[
            pltpu.VMEM((2 * _CH,), jnp.int32),
            pltpu.VMEM((2 * _CH,), jnp.int32),
            pltpu.VMEM((2 * _CH,), jnp.float32),
            pltpu.VMEM((_STG,), jnp.int32),
            pltpu.VMEM((_STG,), jnp.int32),
            pltpu.VMEM((_STG,), jnp.float32),
            pltpu.VMEM((16,), jnp.int32),
            pltpu.SemaphoreType.DMA,
            pltpu.SemaphoreType.DMA,
        ],
        compiler_params=_SC_PARAMS,
    )(dst, src, ew)


def _agg_body(D, G, hp_hbm, srcb_hbm, dstb_hbm, ewb_hbm, tot_hbm,
              neigh_hbm, idxv, dlv, ewv, rows0, rows1, acc, totv,
              sm0, sm1, sg0, sg1):
    w = _wid()
    shift = {64: 6, 128: 7}[G]
    pltpu.sync_copy(tot_hbm.at[w], totv)
    total = totv[pl.ds(0, 16)][0]
    nch = lax.shift_right_logical(total, shift)
    zf = jnp.zeros((16,), jnp.float32)

    idxb = (idxv.at[pl.ds(0, G)], idxv.at[pl.ds(G, G)])
    dlb = (dlv.at[pl.ds(0, G)], dlv.at[pl.ds(G, G)])
    ewb = (ewv.at[pl.ds(0, G)], ewv.at[pl.ds(G, G)])
    rowsb = (rows0, rows1)
    smb = (sm0, sm1)
    sgb = (sg0, sg1)

    def idx_start(ch, b):
        base = w * _CAP + ch * G
        pltpu.make_async_copy(srcb_hbm.at[pl.ds(base, G)], idxb[b],
                              smb[b]).start()

    def idx_wait(b):
        pltpu.make_async_copy(srcb_hbm.at[pl.ds(0, G)], idxb[b],
                              smb[b]).wait()

    def dlew_start(ch, b):
        base = w * _CAP + ch * G
        pltpu.make_async_copy(dstb_hbm.at[pl.ds(base, G)], dlb[b],
                              smb[b]).start()
        pltpu.make_async_copy(ewb_hbm.at[pl.ds(base, G)], ewb[b],
                              smb[b]).start()

    def dlew_wait(b):
        pltpu.make_async_copy(dstb_hbm.at[pl.ds(0, G)], dlb[b],
                              smb[b]).wait()
        pltpu.make_async_copy(ewb_hbm.at[pl.ds(0, G)], ewb[b],
                              smb[b]).wait()

    def gather_start(b):
        pltpu.make_async_copy(hp_hbm.at[idxb[b]], rowsb[b], sgb[b]).start()

    def gather_wait(b):
        pltpu.make_async_copy(hp_hbm.at[idxb[b]], rowsb[b], sgb[b]).wait()

    idx_start(0, 0)
    dlew_start(0, 0)
    idx_start(1, 1)
    dlew_start(1, 1)

    def z(i, c):
        for k in range(8):
            acc[pl.ds((i * 8 + k) * 16, 16)] = zf
        return c

    lax.fori_loop(0, _R * D // 128, z, 0)
    idx_wait(0)
    gather_start(0)

    def process(b):
        nc = D // 16

        def grp(g, c2):
            voff = dlb[b][pl.ds(g * 16, 16)] * D
            vew = ewb[b][pl.ds(g * 16, 16)]
            for j in range(16):
                ej = vew[j]
                r = g * 16 + j
                ab = voff[j]
                mv = [rowsb[b][r, pl.ds(c * 16, 16)] * ej for c in range(nc)]
                cur = [acc[pl.ds(ab + c * 16, 16)] for c in range(nc)]
                for c in range(nc):
                    acc[pl.ds(ab + c * 16, 16)] = jnp.maximum(cur[c], mv[c])
            return c2

        lax.fori_loop(0, G // 16, grp, 0)

    def pair(i, carry):
        for b in range(2):
            ch = 2 * i + b
            gather_wait(b)

            @pl.when(ch + 1 < nch)
            def _():
                idx_wait(1 - b)
                gather_start(1 - b)

            @pl.when(ch + 2 < nch)
            def _():
                idx_start(ch + 2, b)

            dlew_wait(b)
            process(b)

            @pl.when(ch + 2 < nch)
            def _():
                dlew_start(ch + 2, b)

        return carry

    lax.fori_loop(0, lax.shift_right_logical(nch, 1), pair, 0)
    pltpu.sync_copy(acc, neigh_hbm.at[pl.ds(w * _R * D, _R * D)])


@functools.lru_cache(maxsize=None)
def _make_agg(D):
    G = 128 if D <= 128 else 64
    return pl.kernel(
        functools.partial(_agg_body, D, G),
        out_type=jax.ShapeDtypeStruct((_NP * D,), jnp.float32),
        mesh=_mesh(),
        scratch_types=[
            pltpu.VMEM((2 * G,), jnp.int32),
            pltpu.VMEM((2 * G,), jnp.int32),
            pltpu.VMEM((2 * G,), jnp.float32),
            pltpu.VMEM((G, D), jnp.float32),
            pltpu.VMEM((G, D), jnp.float32),
            pltpu.VMEM((_R * D,), jnp.float32),
            pltpu.VMEM((16,), jnp.int32),
            pltpu.SemaphoreType.DMA,
            pltpu.SemaphoreType.DMA,
            pltpu.SemaphoreType.DMA,
            pltpu.SemaphoreType.DMA,
        ],
        compiler_params=_SC_PARAMS,
    )


def _mm1_body(x_ref, w_ref, b_ref, o_ref):
    y = lax.dot_general(x_ref[...], w_ref[...], (((1,), (1,)), ((), ())),
                        preferred_element_type=jnp.float32)
    o_ref[...] = jnp.maximum(y + b_ref[...], 0.0)


def _mm1(x, W, b):
    n, din = x.shape
    dout = W.shape[0]
    bm = 2048
    return pl.pallas_call(
        _mm1_body,
        grid=(n // bm,),
        in_specs=[pl.BlockSpec((bm, din), lambda i: (i, 0)),
                  pl.BlockSpec((dout, din), lambda i: (0, 0)),
                  pl.BlockSpec((1, dout), lambda i: (0, 0))],
        out_specs=pl.BlockSpec((bm, dout), lambda i: (i, 0)),
        out_shape=jax.ShapeDtypeStruct((n, dout), jnp.float32),
    )(x, W, b.reshape(1, -1))


def _mm2_body(x_ref, n_ref, ws_ref, wn_ref, b_ref, o_ref):
    y = lax.dot_general(x_ref[...], ws_ref[...], (((1,), (1,)), ((), ())),
                        preferred_element_type=jnp.float32)
    y = y + lax.dot_general(n_ref[...], wn_ref[...], (((1,), (1,)), ((), ())),
                            preferred_element_type=jnp.float32)
    o_ref[...] = jnp.maximum(y + b_ref[...], 0.0)


def _mm2mm1_body(x_ref, n_ref, ws_ref, wn_ref, b_ref, wp_ref, bp_ref,
                 o_ref, op_ref):
    y = lax.dot_general(x_ref[...], ws_ref[...], (((1,), (1,)), ((), ())),
                        preferred_element_type=jnp.float32)
    y = y + lax.dot_general(n_ref[...], wn_ref[...], (((1,), (1,)), ((), ())),
                            preferred_element_type=jnp.float32)
    y = jnp.maximum(y + b_ref[...], 0.0)
    o_ref[...] = y
    yp = lax.dot_general(y, wp_ref[...], (((1,), (1,)), ((), ())),
                         preferred_element_type=jnp.float32)
    op_ref[...] = jnp.maximum(yp + bp_ref[...], 0.0)


def _mm2mm1(x, neigh, Ws, Wn, b, Wp, bp):
    n, din = x.shape
    dout = Ws.shape[0]
    dp = Wp.shape[0]
    bm = 2048
    return pl.pallas_call(
        _mm2mm1_body,
        grid=(n // bm,),
        in_specs=[pl.BlockSpec((bm, din), lambda i: (i, 0)),
                  pl.BlockSpec((bm, din), lambda i: (i, 0)),
                  pl.BlockSpec((dout, din), lambda i: (0, 0)),
                  pl.BlockSpec((dout, din), lambda i: (0, 0)),
                  pl.BlockSpec((1, dout), lambda i: (0, 0)),
                  pl.BlockSpec((dp, dout), lambda i: (0, 0)),
                  pl.BlockSpec((1, dp), lambda i: (0, 0))],
        out_specs=[pl.BlockSpec((bm, dout), lambda i: (i, 0)),
                   pl.BlockSpec((bm, dp), lambda i: (i, 0))],
        out_shape=[jax.ShapeDtypeStruct((n, dout), jnp.float32),
                   jax.ShapeDtypeStruct((n, dp), jnp.float32)],
    )(x, neigh, Ws, Wn, b.reshape(1, -1), Wp, bp.reshape(1, -1))


def _mm2(x, neigh, Ws, Wn, b):
    n, din = x.shape
    dout = Ws.shape[0]
    bm = 2048
    return pl.pallas_call(
        _mm2_body,
        grid=(n // bm,),
        in_specs=[pl.BlockSpec((bm, din), lambda i: (i, 0)),
                  pl.BlockSpec((bm, din), lambda i: (i, 0)),
                  pl.BlockSpec((dout, din), lambda i: (0, 0)),
                  pl.BlockSpec((dout, din), lambda i: (0, 0)),
                  pl.BlockSpec((1, dout), lambda i: (0, 0))],
        out_specs=pl.BlockSpec((bm, dout), lambda i: (i, 0)),
        out_shape=jax.ShapeDtypeStruct((n, dout), jnp.float32),
    )(x, neigh, Ws, Wn, b.reshape(1, -1))


def kernel(feat, edge_index, edge_weight,
           Wp1, bp1, Ws1, Wn1, b1,
           Wp2, bp2, Ws2, Wn2, b2,
           Wp3, bp3, Ws3, Wn3, b3):
    src = edge_index[0]
    dst = edge_index[1]
    srcb, dstb, ewb, tot = _bucket(dst, src, edge_weight)
    h = jnp.pad(feat, ((0, _NP - _N), (0, 0)))
    hp = _mm1(h, Wp1, bp1)
    layers = ((Ws1, Wn1, b1, Wp2, bp2),
              (Ws2, Wn2, b2, Wp3, bp3),
              (Ws3, Wn3, b3, None, None))
    for Ws, Wn, b, Wpn, bpn in layers:
        d = hp.shape[1]
        neigh = _make_agg(d)(hp, srcb, dstb, ewb, tot).reshape(_NP, d)
        if Wpn is None:
            h = _mm2(h, neigh, Ws, Wn, b)
        else:
            h, hp = _mm2mm1(h, neigh, Ws, Wn, b, Wpn, bpn)
    return h[:_N]
